# baseline (speedup 1.0000x reference)
import functools

import jax
import jax.numpy as jnp
from jax import lax
from jax.experimental import pallas as pl
from jax.experimental.pallas import tpu as pltpu
from jax.experimental.pallas import tpu_sc as plsc

_LANES = 16


def _sc_bow(emb_table, words):
    B, L = words.shape
    V, EMB = emb_table.shape
    mesh = plsc.VectorSubcoreMesh(core_axis_name="c", subcore_axis_name="s")
    NC, NS = mesh.num_cores, mesh.num_subcores
    NW = NC * NS
    GE = 2
    NBUF = 4
    assert B % (NW * GE) == 0
    E = B // NW
    P = E // GE
    R = GE * L
    assert R <= 128
    NCH = EMB // _LANES

    words_g = words.reshape(B // GE, R)

    @functools.partial(
        pl.kernel,
        out_type=jax.ShapeDtypeStruct((B, EMB), jnp.float32),
        mesh=mesh,
        scratch_types=[
            pltpu.VMEM((P, R), jnp.int32),
            pltpu.VMEM((NBUF, R, EMB), jnp.float32),
            pltpu.VMEM((E, EMB), jnp.float32),
        ] + [pltpu.SemaphoreType.DMA] * NBUF,
    )
    def k(table_hbm, words_hbm, out_hbm, idx_v, rows_v, bow_v, *sems):
        wid = lax.axis_index("s") * NC + lax.axis_index("c")
        base = wid * E
        pltpu.sync_copy(words_hbm.at[pl.ds(wid * P, P)], idx_v)

        def start(p, b):
            pltpu.async_copy(table_hbm.at[idx_v.at[p]], rows_v.at[b], sems[b])

        def wait(b):
            pltpu.make_async_copy(
                table_hbm.at[idx_v.at[0]], rows_v.at[b], sems[b]).wait()

        UNROLL = 10
        assert L % UNROLL == 0

        def accum(p, b):
            for g in range(GE):
                def body(i, accs):
                    l0 = g * L + i * UNROLL
                    accs = list(accs)
                    for u in range(UNROLL):
                        for c in range(NCH):
                            accs[c] = accs[c] + rows_v[b, l0 + u,
                                                       pl.ds(c * _LANES, _LANES)]
                    return tuple(accs)
                accs = lax.fori_loop(
                    0, L // UNROLL, body,
                    tuple(jnp.zeros((_LANES,), jnp.float32) for _ in range(NCH)),
                )
                for c in range(NCH):
                    bow_v[p * GE + g, pl.ds(c * _LANES, _LANES)] = accs[c]

        for b in range(NBUF - 1):
            start(b, b)

        @pl.loop(0, P, step=NBUF)
        def _(g0):
            for b in range(NBUF):
                start(jnp.minimum(g0 + b + NBUF - 1, P - 1), (b + NBUF - 1) % NBUF)
                wait(b)
                accum(g0 + b, b)

        for b in range(NBUF - 1):
            wait(b)
        pltpu.sync_copy(bow_v, out_hbm.at[pl.ds(base, E)])

    return k(emb_table, words_g)


def _tc_image_part(image, W, b2d):
    B, FEAT = image.shape
    OUT = W.shape[0]
    BM = 512

    def body(img_ref, w_ref, b_ref, out_ref):
        x = img_ref[...].astype(jnp.bfloat16)
        w = w_ref[:, :FEAT].astype(jnp.bfloat16)
        acc = lax.dot_general(
            w, x, (((1,), (1,)), ((), ())),
            preferred_element_type=jnp.float32,
        )
        out_ref[...] = (acc + b_ref[...]).astype(jnp.bfloat16)

    return pl.pallas_call(
        body,
        grid=(B // BM,),
        in_specs=[
            pl.BlockSpec((BM, FEAT), lambda i: (i, 0)),
            pl.BlockSpec((OUT, W.shape[1]), lambda i: (0, 0)),
            pl.BlockSpec((OUT, 1), lambda i: (0, 0)),
        ],
        out_specs=pl.BlockSpec((OUT, BM), lambda i: (0, i)),
        out_shape=jax.ShapeDtypeStruct((OUT, B), jnp.bfloat16),
    )(image, W, b2d)


def _tc_bow_part(partial_t, bow, W):
    OUT, B = partial_t.shape
    EMB = bow.shape[1]
    FEAT = W.shape[1] - EMB
    BM = 512

    def body(part_ref, bow_ref, w_ref, out_ref):
        x = bow_ref[...].astype(jnp.bfloat16)
        w = w_ref[:, FEAT:].astype(jnp.bfloat16)
        acc = lax.dot_general(
            w, x, (((1,), (1,)), ((), ())),
            preferred_element_type=jnp.float32,
        )
        out_ref[...] = acc + part_ref[...].astype(jnp.float32)

    return pl.pallas_call(
        body,
        grid=(B // BM,),
        in_specs=[
            pl.BlockSpec((OUT, BM), lambda i: (0, i)),
            pl.BlockSpec((BM, EMB), lambda i: (i, 0)),
            pl.BlockSpec((OUT, W.shape[1]), lambda i: (0, 0)),
        ],
        out_specs=pl.BlockSpec((OUT, BM), lambda i: (0, i)),
        out_shape=jax.ShapeDtypeStruct((OUT, B), jnp.float32),
    )(partial_t, bow, W)


def kernel(words, image, emb_table, W, b):
    partial_t = _tc_image_part(image, W, b.reshape(-1, 1))
    bow = _sc_bow(emb_table, words)
    return _tc_bow_part(partial_t, bow, W).T


# speedup vs baseline: 1.0186x; 1.0186x over previous
import functools

import jax
import jax.numpy as jnp
from jax import lax
from jax.experimental import pallas as pl
from jax.experimental.pallas import tpu as pltpu
from jax.experimental.pallas import tpu_sc as plsc

_LANES = 16


def _sc_bow(emb_table, words):
    B, L = words.shape
    V, EMB = emb_table.shape
    mesh = plsc.VectorSubcoreMesh(core_axis_name="c", subcore_axis_name="s")
    NC, NS = mesh.num_cores, mesh.num_subcores
    NW = NC * NS
    GE = 2
    NBUF = 4
    assert B % (NW * GE) == 0
    E = B // NW
    P = E // GE
    R = GE * L
    assert R <= 128
    NCH = EMB // _LANES

    words_g = words.reshape(B // GE, R)

    @functools.partial(
        pl.kernel,
        out_type=jax.ShapeDtypeStruct((B, EMB), jnp.float32),
        mesh=mesh,
        scratch_types=[
            pltpu.VMEM((P, R), jnp.int32),
            pltpu.VMEM((NBUF, R, EMB), jnp.float32),
            pltpu.VMEM((E, EMB), jnp.float32),
        ] + [pltpu.SemaphoreType.DMA] * NBUF,
    )
    def k(table_hbm, words_hbm, out_hbm, idx_v, rows_v, bow_v, *sems):
        wid = lax.axis_index("s") * NC + lax.axis_index("c")
        base = wid * E
        pltpu.sync_copy(words_hbm.at[pl.ds(wid * P, P)], idx_v)

        def start(p, b):
            pltpu.async_copy(table_hbm.at[idx_v.at[p]], rows_v.at[b], sems[b])

        def wait(b):
            pltpu.make_async_copy(
                table_hbm.at[idx_v.at[0]], rows_v.at[b], sems[b]).wait()

        UNROLL = 5
        assert L % UNROLL == 0

        def accum(p, b):
            for g in range(GE):
                def body(i, accs):
                    l0 = g * L + i * UNROLL
                    accs = list(accs)
                    for u in range(UNROLL):
                        for c in range(NCH):
                            accs[c] = accs[c] + rows_v[b, l0 + u,
                                                       pl.ds(c * _LANES, _LANES)]
                    return tuple(accs)
                accs = lax.fori_loop(
                    0, L // UNROLL, body,
                    tuple(jnp.zeros((_LANES,), jnp.float32) for _ in range(NCH)),
                )
                for c in range(NCH):
                    bow_v[p * GE + g, pl.ds(c * _LANES, _LANES)] = accs[c]

        for b in range(NBUF - 1):
            start(b, b)

        @pl.loop(0, P, step=NBUF)
        def _(g0):
            for b in range(NBUF):
                start(jnp.minimum(g0 + b + NBUF - 1, P - 1), (b + NBUF - 1) % NBUF)
                wait(b)
                accum(g0 + b, b)

        for b in range(NBUF - 1):
            wait(b)
        pltpu.sync_copy(bow_v, out_hbm.at[pl.ds(base, E)])

    return k(emb_table, words_g)


def _tc_image_part(image, W, b2d):
    B, FEAT = image.shape
    OUT = W.shape[0]
    BM = 512

    def body(img_ref, w_ref, b_ref, out_ref):
        x = img_ref[...].astype(jnp.bfloat16)
        w = w_ref[:, :FEAT].astype(jnp.bfloat16)
        acc = lax.dot_general(
            w, x, (((1,), (1,)), ((), ())),
            preferred_element_type=jnp.float32,
        )
        out_ref[...] = (acc + b_ref[...]).astype(jnp.bfloat16)

    return pl.pallas_call(
        body,
        grid=(B // BM,),
        in_specs=[
            pl.BlockSpec((BM, FEAT), lambda i: (i, 0)),
            pl.BlockSpec((OUT, W.shape[1]), lambda i: (0, 0)),
            pl.BlockSpec((OUT, 1), lambda i: (0, 0)),
        ],
        out_specs=pl.BlockSpec((OUT, BM), lambda i: (0, i)),
        out_shape=jax.ShapeDtypeStruct((OUT, B), jnp.bfloat16),
    )(image, W, b2d)


def _tc_bow_part(partial_t, bow, W):
    OUT, B = partial_t.shape
    EMB = bow.shape[1]
    FEAT = W.shape[1] - EMB
    BM = 512

    def body(part_ref, bow_ref, w_ref, out_ref):
        x = bow_ref[...].astype(jnp.bfloat16)
        w = w_ref[:, FEAT:].astype(jnp.bfloat16)
        acc = lax.dot_general(
            w, x, (((1,), (1,)), ((), ())),
            preferred_element_type=jnp.float32,
        )
        out_ref[...] = acc + part_ref[...].astype(jnp.float32)

    return pl.pallas_call(
        body,
        grid=(B // BM,),
        in_specs=[
            pl.BlockSpec((OUT, BM), lambda i: (0, i)),
            pl.BlockSpec((BM, EMB), lambda i: (i, 0)),
            pl.BlockSpec((OUT, W.shape[1]), lambda i: (0, 0)),
        ],
        out_specs=pl.BlockSpec((OUT, BM), lambda i: (0, i)),
        out_shape=jax.ShapeDtypeStruct((OUT, B), jnp.float32),
    )(partial_t, bow, W)


def kernel(words, image, emb_table, W, b):
    partial_t = _tc_image_part(image, W, b.reshape(-1, 1))
    bow = _sc_bow(emb_table, words)
    return _tc_bow_part(partial_t, bow, W).T
